# Initial kernel scaffold; baseline (speedup 1.0000x reference)
import jax
import jax.numpy as jnp
from jax import lax
from jax.experimental import pallas as pl
from jax.experimental.pallas import tpu as pltpu
from jax.experimental.pallas import tpu_sc as plsc

_NC = 2
_NS = 16
_NW = _NC * _NS

_B, _L = 4096, 200
_N = _B * _L
_D = 21
_PER_W = _N // _NW
_CHUNK = 3200
_NCHUNK = _PER_W // _CHUNK


def _body(idx_hbm, table_hbm, out_hbm, idx_v, rows_v, sem):
    wid = lax.axis_index("s") * _NC + lax.axis_index("c")
    base = wid * _PER_W

    def step(i, carry):
        start = base + i * _CHUNK
        pltpu.sync_copy(idx_hbm.at[pl.ds(start, _CHUNK)], idx_v)
        pltpu.async_copy(table_hbm.at[idx_v], rows_v, sem).wait()
        pltpu.sync_copy(rows_v, out_hbm.at[pl.ds(start, _CHUNK)])
        return carry

    lax.fori_loop(0, _NCHUNK, step, 0)


def kernel(sequence_indices, table):
    idx_flat = sequence_indices.reshape(_N)
    mesh = plsc.VectorSubcoreMesh(
        core_axis_name="c", subcore_axis_name="s",
        num_cores=_NC, num_subcores=_NS,
    )
    k = pl.kernel(
        _body,
        out_type=jax.ShapeDtypeStruct((_N, _D), jnp.float32),
        mesh=mesh,
        scratch_types=[
            pltpu.VMEM((_CHUNK,), jnp.int32),
            pltpu.VMEM((_CHUNK, _D), jnp.float32),
            pltpu.SemaphoreType.DMA,
        ],
    )
    out = k(idx_flat, table)
    return out.reshape(_B, _L, _D)


# speedup vs baseline: 1.3190x; 1.3190x over previous
import jax
import jax.numpy as jnp
from jax import lax
from jax.experimental import pallas as pl
from jax.experimental.pallas import tpu as pltpu
from jax.experimental.pallas import tpu_sc as plsc

_NC = 2
_NS = 16
_NW = _NC * _NS

_B, _L = 4096, 200
_N = _B * _L
_V = 21
_D = 21
_DP = 24
_IW = 128
_PER_W = _N // _NW
_TILES_W = _PER_W // _IW


def _body(idx_hbm, table_hbm, out_hbm, idx_v, rows_v, sem):
    wid = lax.axis_index("s") * _NC + lax.axis_index("c")

    def step(i, carry):
        off = wid * _PER_W + i * _IW
        pltpu.sync_copy(idx_hbm.at[pl.ds(off, _IW)], idx_v)
        pltpu.async_copy(table_hbm.at[idx_v], rows_v, sem).wait()
        pltpu.sync_copy(rows_v, out_hbm.at[pl.ds(off, _IW)])
        return carry

    lax.fori_loop(0, _TILES_W, step, 0)


def kernel(sequence_indices, table):
    idx_flat = sequence_indices.reshape(_N)
    table_padded = jnp.pad(table, ((0, 0), (0, _DP - _D)))
    mesh = plsc.VectorSubcoreMesh(
        core_axis_name="c", subcore_axis_name="s",
        num_cores=_NC, num_subcores=_NS,
    )
    k = pl.kernel(
        _body,
        out_type=jax.ShapeDtypeStruct((_N, _DP), jnp.float32),
        mesh=mesh,
        scratch_types=[
            pltpu.VMEM((_IW,), jnp.int32),
            pltpu.VMEM((_IW, _DP), jnp.float32),
            pltpu.SemaphoreType.DMA,
        ],
        compiler_params=pltpu.CompilerParams(use_tc_tiling_on_sc=False),
    )
    out = k(idx_flat, table_padded)
    return out[:, :_D].reshape(_B, _L, _D)
